# baseline (speedup 1.0000x reference)
import functools

import jax
import jax.numpy as jnp
from jax import lax
from jax.experimental import pallas as pl
from jax.experimental.pallas import tpu as pltpu
from jax.experimental.pallas import tpu_sc as plsc

_F32 = jnp.float32
_PI = 3.141592653589793

_C = 96
_H = 384
_W = 384
_NW = 32
_CQ = _C // 4

_HT2 = 32
_T = 3 * _HT2
_HBLK = 16
_CC = 8


def _sc_rsqrt(x):
    i = plsc.bitcast(x, jnp.int32)
    i = jnp.int32(0x5F3759DF) - lax.shift_right_logical(i, 1)
    y = plsc.bitcast(i, _F32)
    for _ in range(2):
        y = y * (1.5 - 0.5 * x * y * y)
    return y


def _sc_acos(x):
    ax = jnp.minimum(jnp.abs(x), 1.0)
    p = jnp.full((16,), -0.0187293, _F32)
    for c in (0.0742610, -0.2121144, 1.5707288):
        p = p * ax + jnp.float32(c)
    om = 1.0 - ax
    root = om * _sc_rsqrt(jnp.maximum(om, jnp.float32(1e-37)))
    r = root * p
    return jnp.where(x < 0.0, jnp.float32(_PI) - r, r)


def _sc_stats(o3, l3):
    mesh = plsc.VectorSubcoreMesh(core_axis_name="c", subcore_axis_name="s")
    nchunks = _C // _CC

    @functools.partial(
        pl.kernel,
        out_type=jax.ShapeDtypeStruct((_NW, 128), _F32),
        mesh=mesh,
        compiler_params=pltpu.CompilerParams(use_tc_tiling_on_sc=True,
                                             needs_layout_passes=False),
        scratch_types=[
            pltpu.VMEM((2, _CC, 8, 384), _F32),
            pltpu.VMEM((2, _CC, 8, 384), _F32),
            pltpu.VMEM((4, 8, 384), _F32),
            pltpu.VMEM((128,), _F32),
            pltpu.SemaphoreType.DMA,
            pltpu.SemaphoreType.DMA,
        ],
    )
    def body(o_hbm, l_hbm, part_o, obuf, lbuf, sbuf, pbuf, isem, psem):
        wid = lax.axis_index("s") * 2 + lax.axis_index("c")
        h0 = wid * 8

        def in_copies(k, slot):
            row0 = k * _CC
            src_o = o_hbm.at[pl.ds(row0, _CC), pl.ds(h0, 8), pl.ds(0, 384)]
            src_l = l_hbm.at[pl.ds(row0, _CC), pl.ds(h0, 8), pl.ds(0, 384)]
            return (pltpu.make_async_copy(src_o, obuf.at[slot], isem),
                    pltpu.make_async_copy(src_l, lbuf.at[slot], isem))

        def compute(k, slot):
            def jbody(j, _):
                jh = j // 24
                base = (j - jh * 24) * 16

                z = jnp.zeros((16,), _F32)
                n = z
                oo = z
                llv = z
                ls = z
                for c in range(_CC):
                    ov = obuf[slot, c, jh, pl.ds(base, 16)]
                    lv = lbuf[slot, c, jh, pl.ds(base, 16)]
                    n = n + ov * lv
                    oo = oo + ov * ov
                    llv = llv + lv * lv
                    ls = ls + lv
                if k == 0:
                    sbuf[0, jh, pl.ds(base, 16)] = n
                    sbuf[1, jh, pl.ds(base, 16)] = oo
                    sbuf[2, jh, pl.ds(base, 16)] = llv
                    sbuf[3, jh, pl.ds(base, 16)] = ls
                else:
                    plsc.addupdate(sbuf.at[0, jh, pl.ds(base, 16)], n)
                    plsc.addupdate(sbuf.at[1, jh, pl.ds(base, 16)], oo)
                    plsc.addupdate(sbuf.at[2, jh, pl.ds(base, 16)], llv)
                    plsc.addupdate(sbuf.at[3, jh, pl.ds(base, 16)], ls)
                return 0

            lax.fori_loop(0, 192, jbody, 0)

        def angles():
            def jbody(j, carry):
                s, cn = carry
                jh = j // 24
                base = (j - jh * 24) * 16
                n = sbuf[0, jh, pl.ds(base, 16)]
                oo = sbuf[1, jh, pl.ds(base, 16)]
                llv = sbuf[2, jh, pl.ds(base, 16)]
                ls = sbuf[3, jh, pl.ds(base, 16)]
                mask = ls != 0.0
                ratio = jnp.clip(
                    n * _sc_rsqrt(jnp.maximum(oo * llv, jnp.float32(1e-37))),
                    -1.0, 1.0)
                ang = _sc_acos(ratio)
                one = jnp.full((16,), 1.0, _F32)
                z = jnp.zeros((16,), _F32)
                s = s + jnp.where(mask, ang, z)
                cn = cn + jnp.where(mask, one, z)
                return s, cn

            z = jnp.zeros((16,), _F32)
            return lax.fori_loop(0, 192, jbody, (z, z))

        for cp in in_copies(0, 0):
            cp.start()
        for k in range(nchunks):
            slot = k & 1
            if k + 1 < nchunks:
                for cp in in_copies(k + 1, 1 - slot):
                    cp.start()
            for cp in in_copies(k, slot):
                cp.wait()
            compute(k, slot)
        asum, acnt = angles()

        zb = jnp.zeros((16,), _F32)
        for g in range(8):
            pbuf[pl.ds(g * 16, 16)] = zb
        pbuf[pl.ds(0, 16)] = asum
        pbuf[pl.ds(16, 16)] = acnt
        pltpu.make_async_copy(pbuf, part_o.at[wid], psem).start()
        pltpu.make_async_copy(pbuf, part_o.at[wid], psem).wait()

    return body(o3, l3)


def _acos(x):
    ax = jnp.minimum(jnp.abs(x), 1.0)
    p = jnp.float32(-0.0012624911)
    for c in (0.0066700901, -0.0170881256, 0.0308918810, -0.0501743046,
              0.0889789874, -0.2145988016, 1.5707963050):
        p = p * ax + jnp.float32(c)
    r = jnp.sqrt(1.0 - ax) * p
    return jnp.where(x < 0, jnp.float32(_PI) - r, r)


def _tc_body(o_ref, l_ref, out_ref):
    i = pl.program_id(0)
    o = o_ref[0]
    l = l_ref[0]
    num = jnp.sum(o * l, axis=0)
    oo = jnp.sum(o * o, axis=0)
    ll = jnp.sum(l * l, axis=0)
    ls = jnp.sum(l, axis=0)
    mask = ls != 0.0
    den = jnp.sqrt(oo) * jnp.sqrt(ll)
    ratio = jnp.clip(num / jnp.where(mask, den, 1.0), -1.0, 1.0)
    ang = jnp.where(mask, _acos(ratio), 0.0)
    psum = jnp.sum(ang)
    pcnt = jnp.sum(mask.astype(_F32))

    @pl.when(i == 0)
    def _init():
        out_ref[0] = 0.0
        out_ref[1] = 0.0

    out_ref[0] += psum
    out_ref[1] += pcnt


def _finish_body(sc_ref, tc_ref, out_ref):
    a = sc_ref[...]
    asum = jnp.sum(a[:, 0:16])
    acnt = jnp.sum(a[:, 16:32])
    out_ref[0, 0] = (asum + tc_ref[0]) / (acnt + tc_ref[1])


def kernel(outputs, labels):
    b, c, h, w = outputs.shape
    o3 = outputs.reshape(b * c, h, w)
    l3 = labels.reshape(b * c, h, w)
    sc_part = _sc_stats(o3, l3)

    nh0 = (_H - 8 * _HT2) // _HBLK
    nh1 = _H // _HBLK
    h0_off = (8 * _HT2) // _HBLK

    def imap(i):
        in_b1 = i >= nh0
        return (in_b1.astype(jnp.int32), 0,
                jnp.where(in_b1, i - nh0, i + h0_off), 0)

    spec = pl.BlockSpec((1, c, _HBLK, w), imap)
    tc_part = pl.pallas_call(
        _tc_body,
        grid=(nh0 + nh1,),
        in_specs=[spec, spec],
        out_specs=pl.BlockSpec(memory_space=pltpu.SMEM),
        out_shape=jax.ShapeDtypeStruct((2,), _F32),
    )(outputs, labels)

    out = pl.pallas_call(
        _finish_body,
        in_specs=[pl.BlockSpec((_NW, 128), lambda: (0, 0)),
                  pl.BlockSpec(memory_space=pltpu.SMEM)],
        out_specs=pl.BlockSpec(memory_space=pltpu.SMEM),
        out_shape=jax.ShapeDtypeStruct((1, 1), _F32),
    )(sc_part, tc_part)
    return out[0, 0]


# speedup vs baseline: 1.0441x; 1.0441x over previous
import functools

import jax
import jax.numpy as jnp
from jax import lax
from jax.experimental import pallas as pl
from jax.experimental.pallas import tpu as pltpu
from jax.experimental.pallas import tpu_sc as plsc

_F32 = jnp.float32
_PI = 3.141592653589793

_C = 96
_H = 384
_W = 384
_NW = 32
_CQ = _C // 4

_HT2 = 40
_T = 3 * _HT2
_HBLK = 32


def _sc_rsqrt(x):
    i = plsc.bitcast(x, jnp.int32)
    i = jnp.int32(0x5F3759DF) - lax.shift_right_logical(i, 1)
    y = plsc.bitcast(i, _F32)
    for _ in range(2):
        y = y * (1.5 - 0.5 * x * y * y)
    return y


def _sc_acos(x):
    ax = jnp.minimum(jnp.abs(x), 1.0)
    p = jnp.full((16,), -0.0187293, _F32)
    for c in (0.0742610, -0.2121144, 1.5707288):
        p = p * ax + jnp.float32(c)
    om = 1.0 - ax
    root = om * _sc_rsqrt(jnp.maximum(om, jnp.float32(1e-37)))
    r = root * p
    return jnp.where(x < 0.0, jnp.float32(_PI) - r, r)


def _sc_stats(o3, l3):
    mesh = plsc.VectorSubcoreMesh(core_axis_name="c", subcore_axis_name="s")
    jbase = _T // _NW
    jrem = _T % _NW

    @functools.partial(
        pl.kernel,
        out_type=jax.ShapeDtypeStruct((_NW, 128), _F32),
        mesh=mesh,
        compiler_params=pltpu.CompilerParams(use_tc_tiling_on_sc=True,
                                             needs_layout_passes=False),
        scratch_types=[
            pltpu.VMEM((2, _CQ, 8, 128), _F32),
            pltpu.VMEM((2, _CQ, 8, 128), _F32),
            pltpu.VMEM((4, 8, 128), _F32),
            pltpu.VMEM((128,), _F32),
            pltpu.SemaphoreType.DMA,
            pltpu.SemaphoreType.DMA,
        ],
    )
    def body(o_hbm, l_hbm, part_o, obuf, lbuf, sbuf, pbuf, isem, psem):
        wid = lax.axis_index("s") * 2 + lax.axis_index("c")
        tile0 = wid * jbase + jnp.minimum(wid, jrem)
        njobs = jbase + (wid < jrem).astype(jnp.int32)

        def in_copies(t, q, slot):
            batch = t // 144
            rem = t - batch * 144
            htile = rem // 3
            wtile = rem - htile * 3
            row0 = batch * _C + q * _CQ
            h0 = htile * 8
            w0 = wtile * 128
            src_o = o_hbm.at[pl.ds(row0, _CQ), pl.ds(h0, 8), pl.ds(w0, 128)]
            src_l = l_hbm.at[pl.ds(row0, _CQ), pl.ds(h0, 8), pl.ds(w0, 128)]
            return (pltpu.make_async_copy(src_o, obuf.at[slot], isem),
                    pltpu.make_async_copy(src_l, lbuf.at[slot], isem))

        def compute(q, slot):
            def jbody(j, _):
                jh = j // 8
                base = (j - jh * 8) * 16

                z = jnp.zeros((16,), _F32)
                n = z
                oo = z
                llv = z
                ls = z
                for c in range(_CQ):
                    ov = obuf[slot, c, jh, pl.ds(base, 16)]
                    lv = lbuf[slot, c, jh, pl.ds(base, 16)]
                    n = n + ov * lv
                    oo = oo + ov * ov
                    llv = llv + lv * lv
                    ls = ls + lv
                if q == 0:
                    sbuf[0, jh, pl.ds(base, 16)] = n
                    sbuf[1, jh, pl.ds(base, 16)] = oo
                    sbuf[2, jh, pl.ds(base, 16)] = llv
                    sbuf[3, jh, pl.ds(base, 16)] = ls
                else:
                    plsc.addupdate(sbuf.at[0, jh, pl.ds(base, 16)], n)
                    plsc.addupdate(sbuf.at[1, jh, pl.ds(base, 16)], oo)
                    plsc.addupdate(sbuf.at[2, jh, pl.ds(base, 16)], llv)
                    plsc.addupdate(sbuf.at[3, jh, pl.ds(base, 16)], ls)
                return 0

            lax.fori_loop(0, 64, jbody, 0)

        def angles(asum, acnt):
            def jbody(j, carry):
                s, cn = carry
                jh = j // 8
                base = (j - jh * 8) * 16
                n = sbuf[0, jh, pl.ds(base, 16)]
                oo = sbuf[1, jh, pl.ds(base, 16)]
                llv = sbuf[2, jh, pl.ds(base, 16)]
                ls = sbuf[3, jh, pl.ds(base, 16)]
                mask = ls != 0.0
                ratio = jnp.clip(
                    n * _sc_rsqrt(jnp.maximum(oo * llv, jnp.float32(1e-37))),
                    -1.0, 1.0)
                ang = _sc_acos(ratio)
                one = jnp.full((16,), 1.0, _F32)
                z = jnp.zeros((16,), _F32)
                s = s + jnp.where(mask, ang, z)
                cn = cn + jnp.where(mask, one, z)
                return s, cn

            return lax.fori_loop(0, 64, jbody, (asum, acnt))

        def do_job(t, start_next_job, asum, acnt):
            for q in range(4):
                slot = q & 1
                if q < 3:
                    for cp in in_copies(t, q + 1, 1 - slot):
                        cp.start()
                elif start_next_job:
                    for cp in in_copies(t + 1, 0, 1 - slot):
                        cp.start()
                for cp in in_copies(t, q, slot):
                    cp.wait()
                compute(q, slot)
            return angles(asum, acnt)

        for cp in in_copies(tile0, 0, 0):
            cp.start()

        z = jnp.zeros((16,), _F32)

        def loop_body(job, carry):
            return do_job(tile0 + job, True, *carry)

        asum, acnt = lax.fori_loop(0, njobs - 1, loop_body, (z, z))
        asum, acnt = do_job(tile0 + njobs - 1, False, asum, acnt)

        zb = jnp.zeros((16,), _F32)
        for g in range(8):
            pbuf[pl.ds(g * 16, 16)] = zb
        pbuf[pl.ds(0, 16)] = asum
        pbuf[pl.ds(16, 16)] = acnt
        pltpu.make_async_copy(pbuf, part_o.at[wid], psem).start()
        pltpu.make_async_copy(pbuf, part_o.at[wid], psem).wait()

    return body(o3, l3)


def _acos(x):
    ax = jnp.minimum(jnp.abs(x), 1.0)
    p = jnp.float32(-0.0012624911)
    for c in (0.0066700901, -0.0170881256, 0.0308918810, -0.0501743046,
              0.0889789874, -0.2145988016, 1.5707963050):
        p = p * ax + jnp.float32(c)
    r = jnp.sqrt(1.0 - ax) * p
    return jnp.where(x < 0, jnp.float32(_PI) - r, r)


def _tc_body(o_ref, l_ref, out_ref):
    i = pl.program_id(0)
    o = o_ref[0]
    l = l_ref[0]
    num = jnp.sum(o * l, axis=0)
    oo = jnp.sum(o * o, axis=0)
    ll = jnp.sum(l * l, axis=0)
    ls = jnp.sum(l, axis=0)
    mask = ls != 0.0
    den = jnp.sqrt(oo) * jnp.sqrt(ll)
    ratio = jnp.clip(num / jnp.where(mask, den, 1.0), -1.0, 1.0)
    ang = jnp.where(mask, _acos(ratio), 0.0)
    psum = jnp.sum(ang)
    pcnt = jnp.sum(mask.astype(_F32))

    @pl.when(i == 0)
    def _init():
        out_ref[0] = 0.0
        out_ref[1] = 0.0

    out_ref[0] += psum
    out_ref[1] += pcnt


def _finish_body(sc_ref, tc_ref, out_ref):
    a = sc_ref[...]
    asum = jnp.sum(a[:, 0:16])
    acnt = jnp.sum(a[:, 16:32])
    out_ref[0, 0] = (asum + tc_ref[0]) / (acnt + tc_ref[1])


def kernel(outputs, labels):
    b, c, h, w = outputs.shape
    o3 = outputs.reshape(b * c, h, w)
    l3 = labels.reshape(b * c, h, w)
    sc_part = _sc_stats(o3, l3)

    nh0 = (_H - 8 * _HT2) // _HBLK
    nh1 = _H // _HBLK
    h0_off = (8 * _HT2) // _HBLK

    def imap(i):
        in_b1 = i >= nh0
        return (in_b1.astype(jnp.int32), 0,
                jnp.where(in_b1, i - nh0, i + h0_off), 0)

    spec = pl.BlockSpec((1, c, _HBLK, w), imap)
    tc_part = pl.pallas_call(
        _tc_body,
        grid=(nh0 + nh1,),
        in_specs=[spec, spec],
        out_specs=pl.BlockSpec(memory_space=pltpu.SMEM),
        out_shape=jax.ShapeDtypeStruct((2,), _F32),
    )(outputs, labels)

    out = pl.pallas_call(
        _finish_body,
        in_specs=[pl.BlockSpec((_NW, 128), lambda: (0, 0)),
                  pl.BlockSpec(memory_space=pltpu.SMEM)],
        out_specs=pl.BlockSpec(memory_space=pltpu.SMEM),
        out_shape=jax.ShapeDtypeStruct((1, 1), _F32),
    )(sc_part, tc_part)
    return out[0, 0]
